# baseline (speedup 1.0000x reference)
import functools

import jax
import jax.numpy as jnp
from jax import lax
from jax.experimental import pallas as pl
from jax.experimental.pallas import tpu as pltpu
from jax.experimental.pallas import tpu_sc as plsc

CLS = 100
F = 128
B = 16384
LAMBDA = 0.1

NC = 2
NS = 16
L = 16
NW = NC * NS
RT = B // NW
CHUNK = 128
NCH = RT // CHUNK
CPAD = 128
CPT = CPAD // NS
XW = 2 * L


def _sc_body(f_hbm, t_hbm, s_out, x_out,
             fbuf, tbuf, idx, xbuf, obuf, ob2, sacc, xacc, sem):
    cid = lax.axis_index("c")
    sid = lax.axis_index("s")
    wid = cid * NS + sid
    base = wid * RT

    zv = jnp.zeros((L,), jnp.float32)
    for i in range(CPT):
        for j in range(F // L):
            obuf[i, pl.ds(j * L, L)] = zv
        for j in range(XW // L):
            ob2[i, pl.ds(j * L, L)] = zv
    pltpu.sync_copy(obuf, sacc.at[pl.ds(sid * CPT, CPT)])
    pltpu.sync_copy(ob2, xacc.at[pl.ds(sid * CPT, CPT)])

    pltpu.sync_copy(t_hbm.at[pl.ds(base, RT)], tbuf)
    copies = [
        pltpu.async_copy(f_hbm.at[pl.ds(base + q * CHUNK, CHUNK)],
                         fbuf.at[q], sem)
        for q in range(NCH)
    ]
    for g in range(RT // L):
        v = tbuf[pl.ds(g * L, L)]
        idx[g // (CHUNK // L), pl.ds((g % (CHUNK // L)) * L, L)] = (
            v.astype(jnp.int32))
    for c in copies:
        c.wait()

    ones = jnp.full((L,), 1.0, jnp.float32)

    for q in range(NCH):
        def body(r, _, q=q):
            fv = fbuf[q, r, pl.ds(0, L)]
            acc = fv * fv
            for j in range(1, F // L):
                fv = fbuf[q, r, pl.ds(j * L, L)]
                acc = acc + fv * fv
            xbuf[q, r, pl.ds(0, L)] = acc
            xbuf[q, r, pl.ds(L, L)] = ones
            return _
        lax.fori_loop(0, CHUNK, body, None)

    plsc.subcore_barrier()

    for q in range(NCH):
        pltpu.sync_copy(fbuf.at[q], sacc.at[idx.at[q]], add=True)
        pltpu.sync_copy(xbuf.at[q], xacc.at[idx.at[q]], add=True)

    plsc.subcore_barrier()

    pltpu.sync_copy(sacc.at[pl.ds(sid * CPT, CPT)], obuf)
    pltpu.sync_copy(obuf, s_out.at[cid, pl.ds(sid * CPT, CPT)])
    pltpu.sync_copy(xacc.at[pl.ds(sid * CPT, CPT)], ob2)
    pltpu.sync_copy(ob2, x_out.at[cid, pl.ds(sid * CPT, CPT)])


_sc_call = functools.partial(
    pl.kernel,
    out_type=[
        jax.ShapeDtypeStruct((NC, CPAD, F), jnp.float32),
        jax.ShapeDtypeStruct((NC, CPAD, XW), jnp.float32),
    ],
    mesh=plsc.VectorSubcoreMesh(core_axis_name="c", subcore_axis_name="s"),
    compiler_params=pltpu.CompilerParams(use_tc_tiling_on_sc=False),
    scratch_types=[
        pltpu.VMEM((NCH, CHUNK, F), jnp.float32),
        pltpu.VMEM((RT,), jnp.float32),
        pltpu.VMEM((NCH, CHUNK), jnp.int32),
        pltpu.VMEM((NCH, CHUNK, XW), jnp.float32),
        pltpu.VMEM((CPT, F), jnp.float32),
        pltpu.VMEM((CPT, XW), jnp.float32),
        pltpu.VMEM_SHARED((CPAD, F), jnp.float32),
        pltpu.VMEM_SHARED((CPAD, XW), jnp.float32),
        pltpu.SemaphoreType.DMA,
    ],
)(_sc_body)


def _combine_body(sp_ref, xp_ref, a_ref, out_ref):
    s = sp_ref[0] + sp_ref[1]
    x = xp_ref[0] + xp_ref[1]
    s2 = jnp.sum(x[:, :L], axis=1)
    cnt = jnp.sum(x[:, L:], axis=1) * (1.0 / L)
    a = a_ref[...]
    adots = jnp.sum(a * s, axis=1)
    asq = jnp.sum(a * a, axis=1)
    good = cnt > 0.0
    contrib = jnp.where(
        good, (s2 - 2.0 * adots) / jnp.where(good, cnt, 1.0) + asq, 0.0)
    out_ref[...] = jnp.full((1, 1), LAMBDA * jnp.sum(contrib) / CLS,
                            dtype=jnp.float32)


@jax.jit
def kernel(feature, _target, anchor):
    s_p, x_p = _sc_call(feature, _target)
    a_pad = jnp.pad(anchor, ((0, CPAD - CLS), (0, 0)))
    out = pl.pallas_call(
        _combine_body,
        out_shape=jax.ShapeDtypeStruct((1, 1), jnp.float32),
    )(s_p, x_p, a_pad)
    return out[0, 0]


# speedup vs baseline: 1.1543x; 1.1543x over previous
import functools

import jax
import jax.numpy as jnp
from jax import lax
from jax.experimental import pallas as pl
from jax.experimental.pallas import tpu as pltpu
from jax.experimental.pallas import tpu_sc as plsc

CLS = 100
F = 128
B = 16384
LAMBDA = 0.1

NC = 2
NS = 16
L = 16
NW = NC * NS
RT = B // NW
CHUNK = 128
NCH = RT // CHUNK
CPAD = 128
CPT = CPAD // NS
XW = 2 * L


def _sc_body(f_hbm, t_hbm, s_out, x_out,
             fbuf, tbuf, idx, xbuf, obuf, ob2, sacc, xacc, sem, sem_s):
    cid = lax.axis_index("c")
    sid = lax.axis_index("s")
    wid = cid * NS + sid
    base = wid * RT

    copies = [
        pltpu.async_copy(f_hbm.at[pl.ds(base + q * CHUNK, CHUNK)],
                         fbuf.at[q], sem)
        for q in range(NCH)
    ]

    zv = jnp.zeros((L,), jnp.float32)
    for i in range(CPT):
        for j in range(F // L):
            obuf[i, pl.ds(j * L, L)] = zv
        for j in range(XW // L):
            ob2[i, pl.ds(j * L, L)] = zv
    pltpu.sync_copy(obuf, sacc.at[pl.ds(sid * CPT, CPT)])
    pltpu.sync_copy(ob2, xacc.at[pl.ds(sid * CPT, CPT)])

    pltpu.sync_copy(t_hbm.at[pl.ds(base, RT)], tbuf)
    for g in range(RT // L):
        v = tbuf[pl.ds(g * L, L)]
        idx[g // (CHUNK // L), pl.ds((g % (CHUNK // L)) * L, L)] = (
            v.astype(jnp.int32))

    plsc.subcore_barrier()

    ones = jnp.full((L,), 1.0, jnp.float32)
    UNROLL = 4
    scatters = []
    for q in range(NCH):
        copies[q].wait()

        def body(i, _, q=q):
            r = i * UNROLL
            for k in range(UNROLL):
                m = []
                for j in range(F // L):
                    fv = fbuf[q, r + k, pl.ds(j * L, L)]
                    m.append(fv * fv)
                while len(m) > 1:
                    m = [a + b for a, b in zip(m[::2], m[1::2])]
                xbuf[q, r + k, pl.ds(0, L)] = m[0]
                xbuf[q, r + k, pl.ds(L, L)] = ones
            return _

        lax.fori_loop(0, CHUNK // UNROLL, body, None)
        scatters.append(
            pltpu.async_copy(fbuf.at[q], sacc.at[idx.at[q]], sem_s, add=True))
        scatters.append(
            pltpu.async_copy(xbuf.at[q], xacc.at[idx.at[q]], sem_s, add=True))

    for c in scatters:
        c.wait()

    plsc.subcore_barrier()

    pltpu.sync_copy(sacc.at[pl.ds(sid * CPT, CPT)], obuf)
    pltpu.sync_copy(obuf, s_out.at[cid, pl.ds(sid * CPT, CPT)])
    pltpu.sync_copy(xacc.at[pl.ds(sid * CPT, CPT)], ob2)
    pltpu.sync_copy(ob2, x_out.at[cid, pl.ds(sid * CPT, CPT)])


_sc_call = functools.partial(
    pl.kernel,
    out_type=[
        jax.ShapeDtypeStruct((NC, CPAD, F), jnp.float32),
        jax.ShapeDtypeStruct((NC, CPAD, XW), jnp.float32),
    ],
    mesh=plsc.VectorSubcoreMesh(core_axis_name="c", subcore_axis_name="s"),
    compiler_params=pltpu.CompilerParams(use_tc_tiling_on_sc=False),
    scratch_types=[
        pltpu.VMEM((NCH, CHUNK, F), jnp.float32),
        pltpu.VMEM((RT,), jnp.float32),
        pltpu.VMEM((NCH, CHUNK), jnp.int32),
        pltpu.VMEM((NCH, CHUNK, XW), jnp.float32),
        pltpu.VMEM((CPT, F), jnp.float32),
        pltpu.VMEM((CPT, XW), jnp.float32),
        pltpu.VMEM_SHARED((CPAD, F), jnp.float32),
        pltpu.VMEM_SHARED((CPAD, XW), jnp.float32),
        pltpu.SemaphoreType.DMA,
        pltpu.SemaphoreType.DMA,
    ],
)(_sc_body)


def _combine_body(sp_ref, xp_ref, a_ref, out_ref):
    s = sp_ref[0] + sp_ref[1]
    x = xp_ref[0] + xp_ref[1]
    s2 = jnp.sum(x[:, :L], axis=1)
    cnt = jnp.sum(x[:, L:], axis=1) * (1.0 / L)
    a = a_ref[...]
    adots = jnp.sum(a * s, axis=1)
    asq = jnp.sum(a * a, axis=1)
    good = cnt > 0.0
    contrib = jnp.where(
        good, (s2 - 2.0 * adots) / jnp.where(good, cnt, 1.0) + asq, 0.0)
    out_ref[...] = jnp.full((1, 1), LAMBDA * jnp.sum(contrib) / CLS,
                            dtype=jnp.float32)


@jax.jit
def kernel(feature, _target, anchor):
    s_p, x_p = _sc_call(feature, _target)
    a_pad = jnp.pad(anchor, ((0, CPAD - CLS), (0, 0)))
    out = pl.pallas_call(
        _combine_body,
        out_shape=jax.ShapeDtypeStruct((1, 1), jnp.float32),
    )(s_p, x_p, a_pad)
    return out[0, 0]


# speedup vs baseline: 1.1852x; 1.0267x over previous
import functools

import jax
import jax.numpy as jnp
from jax import lax
from jax.experimental import pallas as pl
from jax.experimental.pallas import tpu as pltpu
from jax.experimental.pallas import tpu_sc as plsc

CLS = 100
F = 128
B = 16384
LAMBDA = 0.1

NC = 2
NS = 16
L = 16
NW = NC * NS
RT = B // NW
CHUNK = 64
NCH = RT // CHUNK
CPAD = 128
CPT = CPAD // NS
XW = 2 * L


def _sc_body(f_hbm, t_hbm, s_out, x_out,
             fbuf, tbuf, idx, xbuf, obuf, ob2, sacc, xacc, sem, sem_s):
    cid = lax.axis_index("c")
    sid = lax.axis_index("s")
    wid = cid * NS + sid
    base = wid * RT

    copies = [
        pltpu.async_copy(f_hbm.at[pl.ds(base + q * CHUNK, CHUNK)],
                         fbuf.at[q], sem)
        for q in range(NCH)
    ]

    zv = jnp.zeros((L,), jnp.float32)
    for i in range(CPT):
        for j in range(F // L):
            obuf[i, pl.ds(j * L, L)] = zv
        for j in range(XW // L):
            ob2[i, pl.ds(j * L, L)] = zv
    pltpu.sync_copy(obuf, sacc.at[pl.ds(sid * CPT, CPT)])
    pltpu.sync_copy(ob2, xacc.at[pl.ds(sid * CPT, CPT)])

    pltpu.sync_copy(t_hbm.at[pl.ds(base, RT)], tbuf)
    for g in range(RT // L):
        v = tbuf[pl.ds(g * L, L)]
        idx[g // (CHUNK // L), pl.ds((g % (CHUNK // L)) * L, L)] = (
            v.astype(jnp.int32))

    plsc.subcore_barrier()

    ones = jnp.full((L,), 1.0, jnp.float32)
    UNROLL = 8
    scatters = []
    for q in range(NCH):
        copies[q].wait()

        def body(i, _, q=q):
            r = i * UNROLL
            for k in range(UNROLL):
                m = []
                for j in range(F // L):
                    fv = fbuf[q, r + k, pl.ds(j * L, L)]
                    m.append(fv * fv)
                while len(m) > 1:
                    m = [a + b for a, b in zip(m[::2], m[1::2])]
                xbuf[q, r + k, pl.ds(0, L)] = m[0]
                xbuf[q, r + k, pl.ds(L, L)] = ones
            return _

        lax.fori_loop(0, CHUNK // UNROLL, body, None)
        scatters.append(
            pltpu.async_copy(fbuf.at[q], sacc.at[idx.at[q]], sem_s, add=True))
        scatters.append(
            pltpu.async_copy(xbuf.at[q], xacc.at[idx.at[q]], sem_s, add=True))

    for c in scatters:
        c.wait()

    plsc.subcore_barrier()

    pltpu.sync_copy(xacc.at[pl.ds(sid * CPT, CPT)], ob2)
    for i in range(CPT):
        for j in range(XW // L):
            obuf[i, pl.ds(j * L, L)] = ob2[i, pl.ds(j * L, L)]
    pltpu.sync_copy(obuf, x_out.at[cid, pl.ds(sid * CPT, CPT)])
    pltpu.sync_copy(sacc.at[pl.ds(sid * CPT, CPT)], obuf)
    pltpu.sync_copy(obuf, s_out.at[cid, pl.ds(sid * CPT, CPT)])


_sc_call = functools.partial(
    pl.kernel,
    out_type=[
        jax.ShapeDtypeStruct((NC, CPAD, F), jnp.float32),
        jax.ShapeDtypeStruct((NC, CPAD, F), jnp.float32),
    ],
    mesh=plsc.VectorSubcoreMesh(core_axis_name="c", subcore_axis_name="s"),
    compiler_params=pltpu.CompilerParams(use_tc_tiling_on_sc=False),
    scratch_types=[
        pltpu.VMEM((NCH, CHUNK, F), jnp.float32),
        pltpu.VMEM((RT,), jnp.float32),
        pltpu.VMEM((NCH, CHUNK), jnp.int32),
        pltpu.VMEM((NCH, CHUNK, XW), jnp.float32),
        pltpu.VMEM((CPT, F), jnp.float32),
        pltpu.VMEM((CPT, XW), jnp.float32),
        pltpu.VMEM_SHARED((CPAD, F), jnp.float32),
        pltpu.VMEM_SHARED((CPAD, XW), jnp.float32),
        pltpu.SemaphoreType.DMA,
        pltpu.SemaphoreType.DMA,
    ],
)(_sc_body)


def _combine_body(sp_ref, xp_ref, a_ref, out_ref):
    s = sp_ref[0] + sp_ref[1]
    x = xp_ref[0] + xp_ref[1]
    s2 = jnp.sum(x[:, :L], axis=1)
    cnt = jnp.sum(x[:, L:XW], axis=1) * (1.0 / L)
    a = a_ref[...]
    adots = jnp.sum(a * s, axis=1)
    asq = jnp.sum(a * a, axis=1)
    good = cnt > 0.0
    contrib = jnp.where(
        good, (s2 - 2.0 * adots) / jnp.where(good, cnt, 1.0) + asq, 0.0)
    out_ref[...] = jnp.full((1, 1), LAMBDA * jnp.sum(contrib) / CLS,
                            dtype=jnp.float32)


@jax.jit
def kernel(feature, _target, anchor):
    s_p, x_p = _sc_call(feature, _target)
    a_pad = jnp.pad(anchor, ((0, CPAD - CLS), (0, 0)))
    out = pl.pallas_call(
        _combine_body,
        out_shape=jax.ShapeDtypeStruct((1, 1), jnp.float32),
    )(s_p, x_p, a_pad)
    return out[0, 0]


# speedup vs baseline: 1.2222x; 1.0313x over previous
import functools

import jax
import jax.numpy as jnp
from jax import lax
from jax.experimental import pallas as pl
from jax.experimental.pallas import tpu as pltpu
from jax.experimental.pallas import tpu_sc as plsc

CLS = 100
F = 128
B = 16384
LAMBDA = 0.1

NC = 2
NS = 16
L = 16
NW = NC * NS
RT = B // NW
CHUNK = 128
NCH = RT // CHUNK
CPAD = 128
CPT = CPAD // NS
XW = 2 * L


def _sc_body(f_hbm, t_hbm, s_out, x_out,
             fbuf, tbuf, idx, xbuf, obuf, ob2, sacc, xacc, sem, sem_s):
    cid = lax.axis_index("c")
    sid = lax.axis_index("s")
    wid = cid * NS + sid
    base = wid * RT

    copies = [
        pltpu.async_copy(f_hbm.at[pl.ds(base + q * CHUNK, CHUNK)],
                         fbuf.at[q], sem)
        for q in range(NCH)
    ]

    zv = jnp.zeros((L,), jnp.float32)
    for i in range(CPT):
        for j in range(F // L):
            obuf[i, pl.ds(j * L, L)] = zv
        for j in range(XW // L):
            ob2[i, pl.ds(j * L, L)] = zv
    pltpu.sync_copy(obuf, sacc.at[pl.ds(sid * CPT, CPT)])
    pltpu.sync_copy(ob2, xacc.at[pl.ds(sid * CPT, CPT)])

    pltpu.sync_copy(t_hbm.at[pl.ds(base, RT)], tbuf)
    for g in range(RT // L):
        v = tbuf[pl.ds(g * L, L)]
        idx[g // (CHUNK // L), pl.ds((g % (CHUNK // L)) * L, L)] = (
            v.astype(jnp.int32))

    plsc.subcore_barrier()

    ones = jnp.full((L,), 1.0, jnp.float32)
    UNROLL = 8
    scatters = []
    for q in range(NCH):
        copies[q].wait()

        def body(i, _, q=q):
            r = i * UNROLL
            for k in range(UNROLL):
                m = []
                for j in range(F // L):
                    fv = fbuf[q, r + k, pl.ds(j * L, L)]
                    m.append(fv * fv)
                while len(m) > 1:
                    m = [a + b for a, b in zip(m[::2], m[1::2])]
                xbuf[q, r + k, pl.ds(0, L)] = m[0]
                xbuf[q, r + k, pl.ds(L, L)] = ones
            return _

        lax.fori_loop(0, CHUNK // UNROLL, body, None)
        scatters.append(
            pltpu.async_copy(fbuf.at[q], sacc.at[idx.at[q]], sem_s, add=True))
        scatters.append(
            pltpu.async_copy(xbuf.at[q], xacc.at[idx.at[q]], sem_s, add=True))

    for c in scatters:
        c.wait()

    plsc.subcore_barrier()

    pltpu.sync_copy(xacc.at[pl.ds(sid * CPT, CPT)], ob2)
    for i in range(CPT):
        for j in range(XW // L):
            obuf[i, pl.ds(j * L, L)] = ob2[i, pl.ds(j * L, L)]
    pltpu.sync_copy(obuf, x_out.at[cid, pl.ds(sid * CPT, CPT)])
    pltpu.sync_copy(sacc.at[pl.ds(sid * CPT, CPT)], obuf)
    pltpu.sync_copy(obuf, s_out.at[cid, pl.ds(sid * CPT, CPT)])


_sc_call = functools.partial(
    pl.kernel,
    out_type=[
        jax.ShapeDtypeStruct((NC, CPAD, F), jnp.float32),
        jax.ShapeDtypeStruct((NC, CPAD, F), jnp.float32),
    ],
    mesh=plsc.VectorSubcoreMesh(core_axis_name="c", subcore_axis_name="s"),
    compiler_params=pltpu.CompilerParams(use_tc_tiling_on_sc=False),
    scratch_types=[
        pltpu.VMEM((NCH, CHUNK, F), jnp.float32),
        pltpu.VMEM((RT,), jnp.float32),
        pltpu.VMEM((NCH, CHUNK), jnp.int32),
        pltpu.VMEM((NCH, CHUNK, XW), jnp.float32),
        pltpu.VMEM((CPT, F), jnp.float32),
        pltpu.VMEM((CPT, XW), jnp.float32),
        pltpu.VMEM_SHARED((CPAD, F), jnp.float32),
        pltpu.VMEM_SHARED((CPAD, XW), jnp.float32),
        pltpu.SemaphoreType.DMA,
        pltpu.SemaphoreType.DMA,
    ],
)(_sc_body)


def _combine_body(sp_ref, xp_ref, a_ref, out_ref):
    s = sp_ref[0, :CLS, :] + sp_ref[1, :CLS, :]
    x = xp_ref[0, :CLS, :] + xp_ref[1, :CLS, :]
    s2 = jnp.sum(x[:, :L], axis=1)
    cnt = jnp.sum(x[:, L:XW], axis=1) * (1.0 / L)
    a = a_ref[...]
    adots = jnp.sum(a * s, axis=1)
    asq = jnp.sum(a * a, axis=1)
    good = cnt > 0.0
    contrib = jnp.where(
        good, (s2 - 2.0 * adots) / jnp.where(good, cnt, 1.0) + asq, 0.0)
    out_ref[...] = jnp.full((1, 1), LAMBDA * jnp.sum(contrib) / CLS,
                            dtype=jnp.float32)


@jax.jit
def kernel(feature, _target, anchor):
    s_p, x_p = _sc_call(feature, _target)
    out = pl.pallas_call(
        _combine_body,
        out_shape=jax.ShapeDtypeStruct((1, 1), jnp.float32),
    )(s_p, x_p, anchor)
    return out[0, 0]


# speedup vs baseline: 1.3502x; 1.1047x over previous
import functools

import jax
import jax.numpy as jnp
from jax import lax
from jax.experimental import pallas as pl
from jax.experimental.pallas import tpu as pltpu
from jax.experimental.pallas import tpu_sc as plsc

CLS = 100
F = 128
B = 16384
LAMBDA = 0.1

B_SC = 8192
NC = 2
NS = 16
L = 16
NW = NC * NS
RT = B_SC // NW
CHUNK = 128
NCH = RT // CHUNK
CPAD = 128
CPT = CPAD // NS
XW = 2 * L

RB = 2048
NBLK_TC = (B - B_SC) // RB
BLK0_TC = B_SC // RB


def _sc_body(f_hbm, t_hbm, s_out, x_out,
             fbuf, tbuf, idx, xbuf, obuf, ob2, sacc, xacc, sem, sem_s):
    cid = lax.axis_index("c")
    sid = lax.axis_index("s")
    wid = cid * NS + sid
    base = wid * RT

    copies = [
        pltpu.async_copy(f_hbm.at[pl.ds(base + q * CHUNK, CHUNK)],
                         fbuf.at[q], sem)
        for q in range(NCH)
    ]

    zv = jnp.zeros((L,), jnp.float32)
    for i in range(CPT):
        for j in range(F // L):
            obuf[i, pl.ds(j * L, L)] = zv
        for j in range(XW // L):
            ob2[i, pl.ds(j * L, L)] = zv
    pltpu.sync_copy(obuf, sacc.at[pl.ds(sid * CPT, CPT)])
    pltpu.sync_copy(ob2, xacc.at[pl.ds(sid * CPT, CPT)])

    pltpu.sync_copy(t_hbm.at[pl.ds(base, RT)], tbuf)
    for g in range(RT // L):
        v = tbuf[pl.ds(g * L, L)]
        idx[g // (CHUNK // L), pl.ds((g % (CHUNK // L)) * L, L)] = (
            v.astype(jnp.int32))

    plsc.subcore_barrier()

    ones = jnp.full((L,), 1.0, jnp.float32)
    UNROLL = 8
    scatters = []
    for q in range(NCH):
        copies[q].wait()

        def body(i, _, q=q):
            r = i * UNROLL
            for k in range(UNROLL):
                m = []
                for j in range(F // L):
                    fv = fbuf[q, r + k, pl.ds(j * L, L)]
                    m.append(fv * fv)
                while len(m) > 1:
                    m = [a + b for a, b in zip(m[::2], m[1::2])]
                xbuf[q, r + k, pl.ds(0, L)] = m[0]
                xbuf[q, r + k, pl.ds(L, L)] = ones
            return _

        lax.fori_loop(0, CHUNK // UNROLL, body, None)
        scatters.append(
            pltpu.async_copy(fbuf.at[q], sacc.at[idx.at[q]], sem_s, add=True))
        scatters.append(
            pltpu.async_copy(xbuf.at[q], xacc.at[idx.at[q]], sem_s, add=True))

    for c in scatters:
        c.wait()

    plsc.subcore_barrier()

    pltpu.sync_copy(xacc.at[pl.ds(sid * CPT, CPT)], ob2)
    for i in range(CPT):
        for j in range(XW // L):
            obuf[i, pl.ds(j * L, L)] = ob2[i, pl.ds(j * L, L)]
    pltpu.sync_copy(obuf, x_out.at[cid, pl.ds(sid * CPT, CPT)])
    pltpu.sync_copy(sacc.at[pl.ds(sid * CPT, CPT)], obuf)
    pltpu.sync_copy(obuf, s_out.at[cid, pl.ds(sid * CPT, CPT)])


_sc_call = functools.partial(
    pl.kernel,
    out_type=[
        jax.ShapeDtypeStruct((NC, CPAD, F), jnp.float32),
        jax.ShapeDtypeStruct((NC, CPAD, F), jnp.float32),
    ],
    mesh=plsc.VectorSubcoreMesh(core_axis_name="c", subcore_axis_name="s"),
    compiler_params=pltpu.CompilerParams(use_tc_tiling_on_sc=False),
    scratch_types=[
        pltpu.VMEM((NCH, CHUNK, F), jnp.float32),
        pltpu.VMEM((RT,), jnp.float32),
        pltpu.VMEM((NCH, CHUNK), jnp.int32),
        pltpu.VMEM((NCH, CHUNK, XW), jnp.float32),
        pltpu.VMEM((CPT, F), jnp.float32),
        pltpu.VMEM((CPT, XW), jnp.float32),
        pltpu.VMEM_SHARED((CPAD, F), jnp.float32),
        pltpu.VMEM_SHARED((CPAD, XW), jnp.float32),
        pltpu.SemaphoreType.DMA,
        pltpu.SemaphoreType.DMA,
    ],
)(_sc_body)


def _tc_body(t_ref, f_ref, s_ref, v_ref, sacc, s2acc, cntacc):
    b = pl.program_id(0)

    @pl.when(b == 0)
    def _init():
        sacc[...] = jnp.zeros_like(sacc)
        s2acc[...] = jnp.zeros_like(s2acc)
        cntacc[...] = jnp.zeros_like(cntacc)

    fblk = f_ref[...]
    idx = t_ref[0, 0, :].astype(jnp.int32)
    cls_iota = jax.lax.broadcasted_iota(jnp.int32, (RB, CPAD), 1)
    onehot = (idx[:, None] == cls_iota).astype(jnp.float32)

    sacc[...] += jax.lax.dot_general(
        onehot, fblk, (((0,), (0,)), ((), ())),
        preferred_element_type=jnp.float32,
        precision=jax.lax.Precision.HIGHEST)
    rowsq = jnp.sum(fblk * fblk, axis=1)[None, :]
    s2acc[...] += jax.lax.dot_general(
        rowsq, onehot, (((1,), (0,)), ((), ())),
        preferred_element_type=jnp.float32,
        precision=jax.lax.Precision.HIGHEST)
    cntacc[...] += jnp.sum(onehot, axis=0, keepdims=True)

    @pl.when(b == NBLK_TC - 1)
    def _fin():
        s_ref[...] = sacc[...]
        v_ref[...] = jnp.concatenate(
            [s2acc[...], cntacc[...],
             jnp.zeros((6, CPAD), jnp.float32)], axis=0)


def _tc_partial(feature, _target):
    t3 = _target.reshape(B // RB, 1, RB)
    return pl.pallas_call(
        _tc_body,
        grid=(NBLK_TC,),
        in_specs=[
            pl.BlockSpec((1, 1, RB), lambda b: (BLK0_TC + b, 0, 0)),
            pl.BlockSpec((RB, F), lambda b: (BLK0_TC + b, 0)),
        ],
        out_specs=[
            pl.BlockSpec((CPAD, F), lambda b: (0, 0)),
            pl.BlockSpec((8, CPAD), lambda b: (0, 0)),
        ],
        out_shape=[
            jax.ShapeDtypeStruct((CPAD, F), jnp.float32),
            jax.ShapeDtypeStruct((8, CPAD), jnp.float32),
        ],
        scratch_shapes=[
            pltpu.VMEM((CPAD, F), jnp.float32),
            pltpu.VMEM((1, CPAD), jnp.float32),
            pltpu.VMEM((1, CPAD), jnp.float32),
        ],
    )(t3, feature)


def _combine_body(sp_ref, xp_ref, stc_ref, vtc_ref, a_ref, out_ref):
    s = sp_ref[0, :CLS, :] + sp_ref[1, :CLS, :] + stc_ref[:CLS, :]
    x = xp_ref[0, :CLS, :] + xp_ref[1, :CLS, :]
    s2 = jnp.sum(x[:, :L], axis=1) + vtc_ref[0, :CLS]
    cnt = (jnp.sum(x[:, L:XW], axis=1) * (1.0 / L)
           + vtc_ref[1, :CLS])
    a = a_ref[...]
    adots = jnp.sum(a * s, axis=1)
    asq = jnp.sum(a * a, axis=1)
    good = cnt > 0.0
    contrib = jnp.where(
        good, (s2 - 2.0 * adots) / jnp.where(good, cnt, 1.0) + asq, 0.0)
    out_ref[...] = jnp.full((1, 1), LAMBDA * jnp.sum(contrib) / CLS,
                            dtype=jnp.float32)


@jax.jit
def kernel(feature, _target, anchor):
    s_p, x_p = _sc_call(feature, _target)
    s_tc, v_tc = _tc_partial(feature, _target)
    out = pl.pallas_call(
        _combine_body,
        out_shape=jax.ShapeDtypeStruct((1, 1), jnp.float32),
    )(s_p, x_p, s_tc, v_tc, anchor)
    return out[0, 0]


# speedup vs baseline: 1.4133x; 1.0468x over previous
import functools

import jax
import jax.numpy as jnp
from jax import lax
from jax.experimental import pallas as pl
from jax.experimental.pallas import tpu as pltpu
from jax.experimental.pallas import tpu_sc as plsc

CLS = 100
F = 128
B = 16384
LAMBDA = 0.1

B_SC = 6144
NC = 2
NS = 16
L = 16
NW = NC * NS
RT = B_SC // NW
CHUNK = 64
NCH = RT // CHUNK
CPAD = 128
CPT = CPAD // NS
XW = 2 * L

RB = 2048
NBLK_TC = (B - B_SC) // RB
BLK0_TC = B_SC // RB


def _sc_body(f_hbm, t_hbm, s_out, x_out,
             fbuf, tbuf, idx, xbuf, obuf, ob2, sacc, xacc, sem, sem_s):
    cid = lax.axis_index("c")
    sid = lax.axis_index("s")
    wid = cid * NS + sid
    base = wid * RT

    copies = [
        pltpu.async_copy(f_hbm.at[pl.ds(base + q * CHUNK, CHUNK)],
                         fbuf.at[q], sem)
        for q in range(NCH)
    ]

    zv = jnp.zeros((L,), jnp.float32)
    for i in range(CPT):
        for j in range(F // L):
            obuf[i, pl.ds(j * L, L)] = zv
        for j in range(XW // L):
            ob2[i, pl.ds(j * L, L)] = zv
    pltpu.sync_copy(obuf, sacc.at[pl.ds(sid * CPT, CPT)])
    pltpu.sync_copy(ob2, xacc.at[pl.ds(sid * CPT, CPT)])

    pltpu.sync_copy(t_hbm.at[pl.ds(base, RT)], tbuf)
    for g in range(RT // L):
        v = tbuf[pl.ds(g * L, L)]
        idx[g // (CHUNK // L), pl.ds((g % (CHUNK // L)) * L, L)] = (
            v.astype(jnp.int32))

    plsc.subcore_barrier()

    ones = jnp.full((L,), 1.0, jnp.float32)
    UNROLL = 8
    scatters = []
    for q in range(NCH):
        copies[q].wait()

        def body(i, _, q=q):
            r = i * UNROLL
            for k in range(UNROLL):
                m = []
                for j in range(F // L):
                    fv = fbuf[q, r + k, pl.ds(j * L, L)]
                    m.append(fv * fv)
                while len(m) > 1:
                    m = [a + b for a, b in zip(m[::2], m[1::2])]
                xbuf[q, r + k, pl.ds(0, L)] = m[0]
                xbuf[q, r + k, pl.ds(L, L)] = ones
            return _

        lax.fori_loop(0, CHUNK // UNROLL, body, None)
        scatters.append(
            pltpu.async_copy(fbuf.at[q], sacc.at[idx.at[q]], sem_s, add=True))
        scatters.append(
            pltpu.async_copy(xbuf.at[q], xacc.at[idx.at[q]], sem_s, add=True))

    for c in scatters:
        c.wait()

    plsc.subcore_barrier()

    pltpu.sync_copy(xacc.at[pl.ds(sid * CPT, CPT)], ob2)
    for i in range(CPT):
        for j in range(XW // L):
            obuf[i, pl.ds(j * L, L)] = ob2[i, pl.ds(j * L, L)]
    pltpu.sync_copy(obuf, x_out.at[cid, pl.ds(sid * CPT, CPT)])
    pltpu.sync_copy(sacc.at[pl.ds(sid * CPT, CPT)], obuf)
    pltpu.sync_copy(obuf, s_out.at[cid, pl.ds(sid * CPT, CPT)])


_sc_call = functools.partial(
    pl.kernel,
    out_type=[
        jax.ShapeDtypeStruct((NC, CPAD, F), jnp.float32),
        jax.ShapeDtypeStruct((NC, CPAD, F), jnp.float32),
    ],
    mesh=plsc.VectorSubcoreMesh(core_axis_name="c", subcore_axis_name="s"),
    compiler_params=pltpu.CompilerParams(use_tc_tiling_on_sc=False),
    scratch_types=[
        pltpu.VMEM((NCH, CHUNK, F), jnp.float32),
        pltpu.VMEM((RT,), jnp.float32),
        pltpu.VMEM((NCH, CHUNK), jnp.int32),
        pltpu.VMEM((NCH, CHUNK, XW), jnp.float32),
        pltpu.VMEM((CPT, F), jnp.float32),
        pltpu.VMEM((CPT, XW), jnp.float32),
        pltpu.VMEM_SHARED((CPAD, F), jnp.float32),
        pltpu.VMEM_SHARED((CPAD, XW), jnp.float32),
        pltpu.SemaphoreType.DMA,
        pltpu.SemaphoreType.DMA,
    ],
)(_sc_body)


def _tc_body(t_ref, f_ref, s_ref, v_ref, sacc, s2acc, cntacc):
    b = pl.program_id(0)

    @pl.when(b == 0)
    def _init():
        sacc[...] = jnp.zeros_like(sacc)
        s2acc[...] = jnp.zeros_like(s2acc)
        cntacc[...] = jnp.zeros_like(cntacc)

    fblk = f_ref[...]
    idx = t_ref[0, 0, :].astype(jnp.int32)
    cls_iota = jax.lax.broadcasted_iota(jnp.int32, (RB, CPAD), 1)
    onehot = (idx[:, None] == cls_iota).astype(jnp.float32)

    sacc[...] += jax.lax.dot_general(
        onehot, fblk, (((0,), (0,)), ((), ())),
        preferred_element_type=jnp.float32,
        precision=jax.lax.Precision.DEFAULT)
    rowsq = jnp.sum(fblk * fblk, axis=1)[None, :]
    s2acc[...] += jax.lax.dot_general(
        rowsq, onehot, (((1,), (0,)), ((), ())),
        preferred_element_type=jnp.float32,
        precision=jax.lax.Precision.DEFAULT)
    cntacc[...] += jnp.sum(onehot, axis=0, keepdims=True)

    @pl.when(b == NBLK_TC - 1)
    def _fin():
        s_ref[...] = sacc[...]
        v_ref[...] = jnp.concatenate(
            [s2acc[...], cntacc[...],
             jnp.zeros((6, CPAD), jnp.float32)], axis=0)


def _tc_partial(feature, _target):
    t3 = _target.reshape(B // RB, 1, RB)
    return pl.pallas_call(
        _tc_body,
        grid=(NBLK_TC,),
        in_specs=[
            pl.BlockSpec((1, 1, RB), lambda b: (BLK0_TC + b, 0, 0)),
            pl.BlockSpec((RB, F), lambda b: (BLK0_TC + b, 0)),
        ],
        out_specs=[
            pl.BlockSpec((CPAD, F), lambda b: (0, 0)),
            pl.BlockSpec((8, CPAD), lambda b: (0, 0)),
        ],
        out_shape=[
            jax.ShapeDtypeStruct((CPAD, F), jnp.float32),
            jax.ShapeDtypeStruct((8, CPAD), jnp.float32),
        ],
        scratch_shapes=[
            pltpu.VMEM((CPAD, F), jnp.float32),
            pltpu.VMEM((1, CPAD), jnp.float32),
            pltpu.VMEM((1, CPAD), jnp.float32),
        ],
    )(t3, feature)


def _combine_body(sp_ref, xp_ref, stc_ref, vtc_ref, a_ref, out_ref):
    s = sp_ref[0, :CLS, :] + sp_ref[1, :CLS, :] + stc_ref[:CLS, :]
    x = xp_ref[0, :CLS, :] + xp_ref[1, :CLS, :]
    s2 = jnp.sum(x[:, :L], axis=1) + vtc_ref[0, :CLS]
    cnt = (jnp.sum(x[:, L:XW], axis=1) * (1.0 / L)
           + vtc_ref[1, :CLS])
    a = a_ref[...]
    adots = jnp.sum(a * s, axis=1)
    asq = jnp.sum(a * a, axis=1)
    good = cnt > 0.0
    contrib = jnp.where(
        good, (s2 - 2.0 * adots) / jnp.where(good, cnt, 1.0) + asq, 0.0)
    out_ref[...] = jnp.full((1, 1), LAMBDA * jnp.sum(contrib) / CLS,
                            dtype=jnp.float32)


@jax.jit
def kernel(feature, _target, anchor):
    s_p, x_p = _sc_call(feature, _target)
    s_tc, v_tc = _tc_partial(feature, _target)
    out = pl.pallas_call(
        _combine_body,
        out_shape=jax.ShapeDtypeStruct((1, 1), jnp.float32),
    )(s_p, x_p, s_tc, v_tc, anchor)
    return out[0, 0]


# speedup vs baseline: 1.4680x; 1.0387x over previous
import functools

import jax
import jax.numpy as jnp
from jax import lax
from jax.experimental import pallas as pl
from jax.experimental.pallas import tpu as pltpu
from jax.experimental.pallas import tpu_sc as plsc

CLS = 100
F = 128
B = 16384
LAMBDA = 0.1

B_SC = 4096
NC = 2
NS = 16
L = 16
NW = NC * NS
RT = B_SC // NW
CHUNK = 64
NCH = RT // CHUNK
CPAD = 128
CPT = CPAD // NS
XW = 2 * L

RB = 2048
NBLK_TC = (B - B_SC) // RB
BLK0_TC = B_SC // RB


def _sc_body(f_hbm, t_hbm, s_out, x_out,
             fbuf, tbuf, idx, xbuf, obuf, ob2, sacc, xacc, sem, sem_s):
    cid = lax.axis_index("c")
    sid = lax.axis_index("s")
    wid = cid * NS + sid
    base = wid * RT

    copies = [
        pltpu.async_copy(f_hbm.at[pl.ds(base + q * CHUNK, CHUNK)],
                         fbuf.at[q], sem)
        for q in range(NCH)
    ]

    zv = jnp.zeros((L,), jnp.float32)
    for i in range(CPT):
        for j in range(F // L):
            obuf[i, pl.ds(j * L, L)] = zv
        for j in range(XW // L):
            ob2[i, pl.ds(j * L, L)] = zv
    pltpu.sync_copy(obuf, sacc.at[pl.ds(sid * CPT, CPT)])
    pltpu.sync_copy(ob2, xacc.at[pl.ds(sid * CPT, CPT)])

    pltpu.sync_copy(t_hbm.at[pl.ds(base, RT)], tbuf)
    for g in range(RT // L):
        v = tbuf[pl.ds(g * L, L)]
        idx[g // (CHUNK // L), pl.ds((g % (CHUNK // L)) * L, L)] = (
            v.astype(jnp.int32))

    plsc.subcore_barrier()

    ones = jnp.full((L,), 1.0, jnp.float32)
    UNROLL = 8
    scatters = []
    for q in range(NCH):
        copies[q].wait()

        def body(i, _, q=q):
            r = i * UNROLL
            for k in range(UNROLL):
                m = []
                for j in range(F // L):
                    fv = fbuf[q, r + k, pl.ds(j * L, L)]
                    m.append(fv * fv)
                while len(m) > 1:
                    m = [a + b for a, b in zip(m[::2], m[1::2])]
                xbuf[q, r + k, pl.ds(0, L)] = m[0]
                xbuf[q, r + k, pl.ds(L, L)] = ones
            return _

        lax.fori_loop(0, CHUNK // UNROLL, body, None)
        scatters.append(
            pltpu.async_copy(fbuf.at[q], sacc.at[idx.at[q]], sem_s, add=True))
        scatters.append(
            pltpu.async_copy(xbuf.at[q], xacc.at[idx.at[q]], sem_s, add=True))

    for c in scatters:
        c.wait()

    plsc.subcore_barrier()

    pltpu.sync_copy(xacc.at[pl.ds(sid * CPT, CPT)], ob2)
    for i in range(CPT):
        for j in range(XW // L):
            obuf[i, pl.ds(j * L, L)] = ob2[i, pl.ds(j * L, L)]
    pltpu.sync_copy(obuf, x_out.at[cid, pl.ds(sid * CPT, CPT)])
    pltpu.sync_copy(sacc.at[pl.ds(sid * CPT, CPT)], obuf)
    pltpu.sync_copy(obuf, s_out.at[cid, pl.ds(sid * CPT, CPT)])


_sc_call = functools.partial(
    pl.kernel,
    out_type=[
        jax.ShapeDtypeStruct((NC, CPAD, F), jnp.float32),
        jax.ShapeDtypeStruct((NC, CPAD, F), jnp.float32),
    ],
    mesh=plsc.VectorSubcoreMesh(core_axis_name="c", subcore_axis_name="s"),
    compiler_params=pltpu.CompilerParams(use_tc_tiling_on_sc=False),
    scratch_types=[
        pltpu.VMEM((NCH, CHUNK, F), jnp.float32),
        pltpu.VMEM((RT,), jnp.float32),
        pltpu.VMEM((NCH, CHUNK), jnp.int32),
        pltpu.VMEM((NCH, CHUNK, XW), jnp.float32),
        pltpu.VMEM((CPT, F), jnp.float32),
        pltpu.VMEM((CPT, XW), jnp.float32),
        pltpu.VMEM_SHARED((CPAD, F), jnp.float32),
        pltpu.VMEM_SHARED((CPAD, XW), jnp.float32),
        pltpu.SemaphoreType.DMA,
        pltpu.SemaphoreType.DMA,
    ],
)(_sc_body)


def _tc_body(t_ref, f_ref, s_ref, v_ref, sacc, s2acc, cntacc):
    b = pl.program_id(0)

    @pl.when(b == 0)
    def _init():
        sacc[...] = jnp.zeros_like(sacc)
        s2acc[...] = jnp.zeros_like(s2acc)
        cntacc[...] = jnp.zeros_like(cntacc)

    fblk = f_ref[...]
    idx = t_ref[0, 0, :].astype(jnp.int32)
    cls_iota = jax.lax.broadcasted_iota(jnp.int32, (RB, CPAD), 1)
    onehot = (idx[:, None] == cls_iota).astype(jnp.float32)

    sacc[...] += jax.lax.dot_general(
        onehot, fblk, (((0,), (0,)), ((), ())),
        preferred_element_type=jnp.float32,
        precision=jax.lax.Precision.DEFAULT)
    rowsq = jnp.sum(fblk * fblk, axis=1)[None, :]
    s2acc[...] += jax.lax.dot_general(
        rowsq, onehot, (((1,), (0,)), ((), ())),
        preferred_element_type=jnp.float32,
        precision=jax.lax.Precision.DEFAULT)
    cntacc[...] += jnp.sum(onehot, axis=0, keepdims=True)

    @pl.when(b == NBLK_TC - 1)
    def _fin():
        s_ref[...] = sacc[...]
        v_ref[...] = jnp.concatenate(
            [s2acc[...], cntacc[...],
             jnp.zeros((6, CPAD), jnp.float32)], axis=0)


def _tc_partial(feature, _target):
    t3 = _target.reshape(B // RB, 1, RB)
    return pl.pallas_call(
        _tc_body,
        grid=(NBLK_TC,),
        in_specs=[
            pl.BlockSpec((1, 1, RB), lambda b: (BLK0_TC + b, 0, 0)),
            pl.BlockSpec((RB, F), lambda b: (BLK0_TC + b, 0)),
        ],
        out_specs=[
            pl.BlockSpec((CPAD, F), lambda b: (0, 0)),
            pl.BlockSpec((8, CPAD), lambda b: (0, 0)),
        ],
        out_shape=[
            jax.ShapeDtypeStruct((CPAD, F), jnp.float32),
            jax.ShapeDtypeStruct((8, CPAD), jnp.float32),
        ],
        scratch_shapes=[
            pltpu.VMEM((CPAD, F), jnp.float32),
            pltpu.VMEM((1, CPAD), jnp.float32),
            pltpu.VMEM((1, CPAD), jnp.float32),
        ],
    )(t3, feature)


def _combine_body(sp_ref, xp_ref, stc_ref, vtc_ref, a_ref, out_ref):
    s = sp_ref[0, :CLS, :] + sp_ref[1, :CLS, :] + stc_ref[:CLS, :]
    x = xp_ref[0, :CLS, :] + xp_ref[1, :CLS, :]
    s2 = jnp.sum(x[:, :L], axis=1) + vtc_ref[0, :CLS]
    cnt = (jnp.sum(x[:, L:XW], axis=1) * (1.0 / L)
           + vtc_ref[1, :CLS])
    a = a_ref[...]
    adots = jnp.sum(a * s, axis=1)
    asq = jnp.sum(a * a, axis=1)
    good = cnt > 0.0
    contrib = jnp.where(
        good, (s2 - 2.0 * adots) / jnp.where(good, cnt, 1.0) + asq, 0.0)
    out_ref[...] = jnp.full((1, 1), LAMBDA * jnp.sum(contrib) / CLS,
                            dtype=jnp.float32)


@jax.jit
def kernel(feature, _target, anchor):
    s_p, x_p = _sc_call(feature, _target)
    s_tc, v_tc = _tc_partial(feature, _target)
    out = pl.pallas_call(
        _combine_body,
        out_shape=jax.ShapeDtypeStruct((1, 1), jnp.float32),
    )(s_p, x_p, s_tc, v_tc, anchor)
    return out[0, 0]


# speedup vs baseline: 1.4821x; 1.0096x over previous
import functools

import jax
import jax.numpy as jnp
from jax import lax
from jax.experimental import pallas as pl
from jax.experimental.pallas import tpu as pltpu
from jax.experimental.pallas import tpu_sc as plsc

CLS = 100
F = 128
B = 16384
LAMBDA = 0.1

B_SC = 4096
NC = 2
NS = 16
L = 16
NW = NC * NS
RT = B_SC // NW
CHUNK = 64
NCH = RT // CHUNK
CPAD = 128
CPT = CPAD // NS
XW = 2 * L

RB = 4096
NBLK_TC = (B - B_SC) // RB
BLK0_TC = B_SC // RB


def _sc_body(f_hbm, t_hbm, s_out, x_out,
             fbuf, tbuf, idx, xbuf, obuf, ob2, sacc, xacc, sem, sem_s):
    cid = lax.axis_index("c")
    sid = lax.axis_index("s")
    wid = cid * NS + sid
    base = wid * RT

    copies = [
        pltpu.async_copy(f_hbm.at[pl.ds(base + q * CHUNK, CHUNK)],
                         fbuf.at[q], sem)
        for q in range(NCH)
    ]

    zv = jnp.zeros((L,), jnp.float32)
    for i in range(CPT):
        for j in range(F // L):
            obuf[i, pl.ds(j * L, L)] = zv
        for j in range(XW // L):
            ob2[i, pl.ds(j * L, L)] = zv
    pltpu.sync_copy(obuf, sacc.at[pl.ds(sid * CPT, CPT)])
    pltpu.sync_copy(ob2, xacc.at[pl.ds(sid * CPT, CPT)])

    pltpu.sync_copy(t_hbm.at[pl.ds(base, RT)], tbuf)
    for g in range(RT // L):
        v = tbuf[pl.ds(g * L, L)]
        idx[g // (CHUNK // L), pl.ds((g % (CHUNK // L)) * L, L)] = (
            v.astype(jnp.int32))

    plsc.subcore_barrier()

    ones = jnp.full((L,), 1.0, jnp.float32)
    UNROLL = 8
    scatters = []
    for q in range(NCH):
        copies[q].wait()

        def body(i, _, q=q):
            r = i * UNROLL
            for k in range(UNROLL):
                m = []
                for j in range(F // L):
                    fv = fbuf[q, r + k, pl.ds(j * L, L)]
                    m.append(fv * fv)
                while len(m) > 1:
                    m = [a + b for a, b in zip(m[::2], m[1::2])]
                xbuf[q, r + k, pl.ds(0, L)] = m[0]
                xbuf[q, r + k, pl.ds(L, L)] = ones
            return _

        lax.fori_loop(0, CHUNK // UNROLL, body, None)
        scatters.append(
            pltpu.async_copy(fbuf.at[q], sacc.at[idx.at[q]], sem_s, add=True))
        scatters.append(
            pltpu.async_copy(xbuf.at[q], xacc.at[idx.at[q]], sem_s, add=True))

    for c in scatters:
        c.wait()

    plsc.subcore_barrier()

    pltpu.sync_copy(xacc.at[pl.ds(sid * CPT, CPT)], ob2)
    for i in range(CPT):
        for j in range(XW // L):
            obuf[i, pl.ds(j * L, L)] = ob2[i, pl.ds(j * L, L)]
    pltpu.sync_copy(obuf, x_out.at[cid, pl.ds(sid * CPT, CPT)])
    pltpu.sync_copy(sacc.at[pl.ds(sid * CPT, CPT)], obuf)
    pltpu.sync_copy(obuf, s_out.at[cid, pl.ds(sid * CPT, CPT)])


_sc_call = functools.partial(
    pl.kernel,
    out_type=[
        jax.ShapeDtypeStruct((NC, CPAD, F), jnp.float32),
        jax.ShapeDtypeStruct((NC, CPAD, F), jnp.float32),
    ],
    mesh=plsc.VectorSubcoreMesh(core_axis_name="c", subcore_axis_name="s"),
    compiler_params=pltpu.CompilerParams(use_tc_tiling_on_sc=False),
    scratch_types=[
        pltpu.VMEM((NCH, CHUNK, F), jnp.float32),
        pltpu.VMEM((RT,), jnp.float32),
        pltpu.VMEM((NCH, CHUNK), jnp.int32),
        pltpu.VMEM((NCH, CHUNK, XW), jnp.float32),
        pltpu.VMEM((CPT, F), jnp.float32),
        pltpu.VMEM((CPT, XW), jnp.float32),
        pltpu.VMEM_SHARED((CPAD, F), jnp.float32),
        pltpu.VMEM_SHARED((CPAD, XW), jnp.float32),
        pltpu.SemaphoreType.DMA,
        pltpu.SemaphoreType.DMA,
    ],
)(_sc_body)


def _tc_body(t_ref, f_ref, s_ref, v_ref, sacc, s2acc, cntacc):
    b = pl.program_id(0)

    @pl.when(b == 0)
    def _init():
        sacc[...] = jnp.zeros_like(sacc)
        s2acc[...] = jnp.zeros_like(s2acc)
        cntacc[...] = jnp.zeros_like(cntacc)

    fblk = f_ref[...]
    idx = t_ref[0, 0, :].astype(jnp.int32)
    cls_iota = jax.lax.broadcasted_iota(jnp.int32, (RB, CPAD), 1)
    onehot = (idx[:, None] == cls_iota).astype(jnp.float32)

    sacc[...] += jax.lax.dot_general(
        onehot, fblk, (((0,), (0,)), ((), ())),
        preferred_element_type=jnp.float32,
        precision=jax.lax.Precision.DEFAULT)
    rowsq = jnp.sum(fblk * fblk, axis=1)[None, :]
    s2acc[...] += jax.lax.dot_general(
        rowsq, onehot, (((1,), (0,)), ((), ())),
        preferred_element_type=jnp.float32,
        precision=jax.lax.Precision.DEFAULT)
    cntacc[...] += jnp.sum(onehot, axis=0, keepdims=True)

    @pl.when(b == NBLK_TC - 1)
    def _fin():
        s_ref[...] = sacc[...]
        v_ref[...] = jnp.concatenate(
            [s2acc[...], cntacc[...],
             jnp.zeros((6, CPAD), jnp.float32)], axis=0)


def _tc_partial(feature, _target):
    t3 = _target.reshape(B // RB, 1, RB)
    return pl.pallas_call(
        _tc_body,
        grid=(NBLK_TC,),
        in_specs=[
            pl.BlockSpec((1, 1, RB), lambda b: (BLK0_TC + b, 0, 0)),
            pl.BlockSpec((RB, F), lambda b: (BLK0_TC + b, 0)),
        ],
        out_specs=[
            pl.BlockSpec((CPAD, F), lambda b: (0, 0)),
            pl.BlockSpec((8, CPAD), lambda b: (0, 0)),
        ],
        out_shape=[
            jax.ShapeDtypeStruct((CPAD, F), jnp.float32),
            jax.ShapeDtypeStruct((8, CPAD), jnp.float32),
        ],
        scratch_shapes=[
            pltpu.VMEM((CPAD, F), jnp.float32),
            pltpu.VMEM((1, CPAD), jnp.float32),
            pltpu.VMEM((1, CPAD), jnp.float32),
        ],
    )(t3, feature)


def _combine_body(sp_ref, xp_ref, stc_ref, vtc_ref, a_ref, out_ref):
    s = sp_ref[0, :CLS, :] + sp_ref[1, :CLS, :] + stc_ref[:CLS, :]
    x = xp_ref[0, :CLS, :] + xp_ref[1, :CLS, :]
    s2 = jnp.sum(x[:, :L], axis=1) + vtc_ref[0, :CLS]
    cnt = (jnp.sum(x[:, L:XW], axis=1) * (1.0 / L)
           + vtc_ref[1, :CLS])
    a = a_ref[...]
    adots = jnp.sum(a * s, axis=1)
    asq = jnp.sum(a * a, axis=1)
    good = cnt > 0.0
    contrib = jnp.where(
        good, (s2 - 2.0 * adots) / jnp.where(good, cnt, 1.0) + asq, 0.0)
    out_ref[...] = jnp.full((1, 1), LAMBDA * jnp.sum(contrib) / CLS,
                            dtype=jnp.float32)


@jax.jit
def kernel(feature, _target, anchor):
    s_p, x_p = _sc_call(feature, _target)
    s_tc, v_tc = _tc_partial(feature, _target)
    out = pl.pallas_call(
        _combine_body,
        out_shape=jax.ShapeDtypeStruct((1, 1), jnp.float32),
    )(s_p, x_p, s_tc, v_tc, anchor)
    return out[0, 0]
